# baseline (speedup 1.0000x reference)
import jax
import jax.numpy as jnp
from jax.experimental import pallas as pl
from jax.experimental.pallas import tpu as pltpu

BM = 2048
BN = 2048
BK = 256


def _masked_linear_kernel(d_ref, w_ref, m_ref, b_ref, o_ref):
    k = pl.program_id(2)
    w = w_ref[...] * m_ref[...]
    d = d_ref[...].astype(jnp.bfloat16)
    prod = jax.lax.dot_general(
        d, w,
        dimension_numbers=(((1,), (0,)), ((), ())),
        preferred_element_type=jnp.float32,
    )

    @pl.when(k == 0)
    def _init():
        o_ref[...] = prod + b_ref[...]

    @pl.when(k > 0)
    def _acc():
        o_ref[...] += prod


def kernel(data, w_mask, weight, bias):
    M, K = data.shape
    N = weight.shape[0]
    bm, bn, bk = min(BM, M), min(BN, N), min(BK, K)

    wt16 = weight.T.astype(jnp.bfloat16)
    mt16 = w_mask.T.astype(jnp.bfloat16)
    b2 = bias.reshape(1, N)

    grid = (N // bn, M // bm, K // bk)
    return pl.pallas_call(
        _masked_linear_kernel,
        grid=grid,
        in_specs=[
            pl.BlockSpec((bm, bk), lambda j, i, k: (i, k)),
            pl.BlockSpec((bk, bn), lambda j, i, k: (k, j)),
            pl.BlockSpec((bk, bn), lambda j, i, k: (k, j)),
            pl.BlockSpec((1, bn), lambda j, i, k: (0, j)),
        ],
        out_specs=pl.BlockSpec((bm, bn), lambda j, i, k: (i, j)),
        out_shape=jax.ShapeDtypeStruct((M, N), jnp.float32),
        compiler_params=pltpu.CompilerParams(
            dimension_semantics=("parallel", "parallel", "arbitrary"),
        ),
    )(data, wt16, mt16, b2)


# speedup vs baseline: 1.3142x; 1.3142x over previous
import jax
import jax.numpy as jnp
from jax.experimental import pallas as pl
from jax.experimental.pallas import tpu as pltpu

BM = 1024
BN = 1024
BK = 2048


def _masked_linear_kernel(d_ref, w_ref, m_ref, b_ref, o_ref):
    k = pl.program_id(2)
    w = w_ref[...] * m_ref[...]
    d = d_ref[...].astype(jnp.bfloat16)
    prod = jax.lax.dot_general(
        d, w,
        dimension_numbers=(((1,), (0,)), ((), ())),
        preferred_element_type=jnp.float32,
    )

    @pl.when(k == 0)
    def _init():
        o_ref[...] = prod + b_ref[...]

    @pl.when(k > 0)
    def _acc():
        o_ref[...] += prod


def kernel(data, w_mask, weight, bias):
    M, K = data.shape
    N = weight.shape[0]
    bm, bn, bk = min(BM, M), min(BN, N), min(BK, K)

    wt16 = weight.T.astype(jnp.bfloat16)
    mt16 = w_mask.T.astype(jnp.bfloat16)
    b2 = bias.reshape(1, N)

    grid = (N // bn, M // bm, K // bk)
    return pl.pallas_call(
        _masked_linear_kernel,
        grid=grid,
        in_specs=[
            pl.BlockSpec((bm, bk), lambda j, i, k: (i, k)),
            pl.BlockSpec((bk, bn), lambda j, i, k: (k, j)),
            pl.BlockSpec((bk, bn), lambda j, i, k: (k, j)),
            pl.BlockSpec((1, bn), lambda j, i, k: (0, j)),
        ],
        out_specs=pl.BlockSpec((bm, bn), lambda j, i, k: (i, j)),
        out_shape=jax.ShapeDtypeStruct((M, N), jnp.float32),
        compiler_params=pltpu.CompilerParams(
            dimension_semantics=("parallel", "parallel", "arbitrary"),
        ),
    )(data, wt16, mt16, b2)
